# baseline (speedup 1.0000x reference)
import functools

import jax
import jax.numpy as jnp
from jax import lax
from jax.experimental import pallas as pl
from jax.experimental.pallas import tpu as pltpu
from jax.experimental.pallas import tpu_sc as plsc

N = 10000
E = 320000
F = 128
R = 16
NBASES = 4

NC = 2
NS = 16
NTILES = NC * NS
EPT = E // NTILES
K = 125
NBT = EPT // K
C = 8
NCHUNK = NBT // C
NFL = 10
FRPT = N // NFL
ZROWS = 40


def _zr_body(coeff_ref, w_ref, feat_ref, lw_ref, bias_ref, zr_ref, sl_ref,
             wr_all):
    i = pl.program_id(0)

    @pl.when(i == 0)
    def _build_wr_table():
        for rr in range(R):
            wr_all[:, rr * F:(rr + 1) * F] = (
                coeff_ref[rr, 0] * w_ref[0] + coeff_ref[rr, 1] * w_ref[1]
                + coeff_ref[rr, 2] * w_ref[2] + coeff_ref[rr, 3] * w_ref[3])

    wide = jnp.dot(feat_ref[...], wr_all[...],
                   preferred_element_type=jnp.float32)
    for rr in range(R):
        zr_ref[rr] = wide[:, rr * F:(rr + 1) * F]
    sl_ref[...] = bias_ref[...] + jnp.dot(feat_ref[...], lw_ref[...],
                                          preferred_element_type=jnp.float32)


def _build_zr(coeff, W, feat, lw, bias2d):
    bn = 1000
    nb = N // bn
    return pl.pallas_call(
        _zr_body,
        grid=(nb,),
        in_specs=[
            pl.BlockSpec(memory_space=pltpu.SMEM),
            pl.BlockSpec((NBASES, F, F), lambda i: (0, 0, 0)),
            pl.BlockSpec((bn, F), lambda i: (i, 0)),
            pl.BlockSpec((F, F), lambda i: (0, 0)),
            pl.BlockSpec((1, F), lambda i: (0, 0)),
        ],
        out_specs=[
            pl.BlockSpec((R, bn, F), lambda i: (0, i, 0)),
            pl.BlockSpec((bn, F), lambda i: (i, 0)),
        ],
        out_shape=[
            jax.ShapeDtypeStruct((R, N, F), jnp.float32),
            jax.ShapeDtypeStruct((N, F), jnp.float32),
        ],
        scratch_shapes=[pltpu.VMEM((F, R * F), jnp.float32)],
    )(coeff, W, feat, lw, bias2d)


def _sc_agg_body(zr, eiR, etR, out, ridx_v, et_v, dst_v, rows_v,
                 zbuf, acc, sem, isem):
    c = lax.axis_index("c")
    s = lax.axis_index("s")
    wid = c * NS + s
    base = s * FRPT

    def _stage(ch, pc):
        pltpu.async_copy(eiR.at[0].at[wid].at[ch], ridx_v.at[pc], isem)
        pltpu.async_copy(etR.at[wid].at[ch], et_v.at[pc], isem)
        pltpu.async_copy(eiR.at[1].at[wid].at[ch], dst_v.at[pc], isem)

    def _stage_wait(ch, pc):
        pltpu.make_async_copy(eiR.at[0].at[wid].at[ch], ridx_v.at[pc],
                              isem).wait()
        pltpu.make_async_copy(etR.at[wid].at[ch], et_v.at[pc], isem).wait()
        pltpu.make_async_copy(eiR.at[1].at[wid].at[ch], dst_v.at[pc],
                              isem).wait()
        ntail = K - (K // 16) * 16
        tail0 = K - 16
        ovl = 16 - ntail
        tmask = lax.iota(jnp.int32, 16) >= ovl
        def _cvt(i, cy):
            for j in range(K // 16):
                sl = pl.ds(j * 16, 16)
                ridx_v[pc, i, sl] = ridx_v[pc, i, sl] + et_v[pc, i, sl] * N
            if ntail:
                sl = pl.ds(tail0, 16)
                add = jnp.where(tmask, et_v[pc, i, sl] * N, 0)
                ridx_v[pc, i, sl] = ridx_v[pc, i, sl] + add
            return cy
        lax.fori_loop(0, C, _cvt, 0)

    _stage(0, 0)

    @pl.when(s < NFL)
    def _zero():
        zero16 = jnp.zeros((16,), jnp.float32)
        def _zrow(i, carry):
            for j in range(F // 16):
                zbuf[i, pl.ds(j * 16, 16)] = zero16
            return carry
        lax.fori_loop(0, ZROWS, _zrow, 0)
        for j in range(FRPT // ZROWS):
            pltpu.sync_copy(zbuf, acc.at[pl.ds(base + j * ZROWS, ZROWS)])

    _stage_wait(0, 0)
    plsc.subcore_barrier()

    def _chunk(ch, carry):
        pc = lax.rem(ch, 2)
        qc = lax.rem(ch + 1, 2)

        @pl.when(ch + 1 < NCHUNK)
        def _prefetch_idx():
            _stage(ch + 1, qc)

        pltpu.async_copy(zr.at[ridx_v.at[pc].at[0]], rows_v.at[0], sem.at[0])

        def _edge(i, cy):
            p = lax.rem(i, 2)
            q = lax.rem(i + 1, 2)

            @pl.when(i + 1 < C)
            def _prefetch():
                pltpu.async_copy(zr.at[ridx_v.at[pc].at[i + 1]], rows_v.at[q],
                                 sem.at[q])
            pltpu.make_async_copy(zr.at[ridx_v.at[pc].at[i]], rows_v.at[p],
                                  sem.at[p]).wait()
            pltpu.sync_copy(rows_v.at[p], acc.at[dst_v.at[pc].at[i]], add=True)
            return cy
        lax.fori_loop(0, C, _edge, 0)

        @pl.when(ch + 1 < NCHUNK)
        def _wait_idx():
            _stage_wait(ch + 1, qc)
        return carry
    lax.fori_loop(0, NCHUNK, _chunk, 0)
    plsc.subcore_barrier()

    @pl.when(s < NFL)
    def _flush():
        pltpu.sync_copy(acc.at[pl.ds(base, FRPT)],
                        out.at[c].at[pl.ds(base, FRPT)])


def _sc_agg(zr2, eiR, etR):
    mesh = plsc.VectorSubcoreMesh(core_axis_name="c", subcore_axis_name="s")
    fn = functools.partial(
        pl.kernel,
        mesh=mesh,
        out_type=jax.ShapeDtypeStruct((NC, N, F), jnp.float32),
        scratch_types=[
            pltpu.VMEM((2, C, K), jnp.int32),
            pltpu.VMEM((2, C, K), jnp.int32),
            pltpu.VMEM((2, C, K), jnp.int32),
            pltpu.VMEM((2, K, F), jnp.float32),
            pltpu.VMEM((ZROWS, F), jnp.float32),
            pltpu.VMEM_SHARED((N, F), jnp.float32),
            pltpu.SemaphoreType.DMA((2,)),
            pltpu.SemaphoreType.DMA,
        ],
    )(_sc_agg_body)
    return fn(zr2, eiR, etR)


def _combine_body(p0_ref, p1_ref, sl_ref, out_ref):
    out_ref[...] = p0_ref[0] + p1_ref[0] + sl_ref[...]


def _combine(parts, sl):
    bn = 2000
    return pl.pallas_call(
        _combine_body,
        grid=(N // bn,),
        in_specs=[
            pl.BlockSpec((1, bn, F), lambda i: (0, i, 0)),
            pl.BlockSpec((1, bn, F), lambda i: (1, i, 0)),
            pl.BlockSpec((bn, F), lambda i: (i, 0)),
        ],
        out_specs=pl.BlockSpec((bn, F), lambda i: (i, 0)),
        out_shape=jax.ShapeDtypeStruct((N, F), jnp.float32),
    )(parts, parts, sl)


def kernel(feat, edge_index, etypes, W, coeff, h_bias, loop_weight):
    zr3, sl = _build_zr(coeff, W, feat, loop_weight,
                        h_bias.reshape(1, F))
    zr2 = zr3.reshape(R * N, F)
    eiR = edge_index.reshape(2, NTILES, NCHUNK, C, K)
    etR = etypes.reshape(NTILES, NCHUNK, C, K)
    parts = _sc_agg(zr2, eiR, etR)
    return _combine(parts, sl)


# speedup vs baseline: 1.0068x; 1.0068x over previous
import functools

import jax
import jax.numpy as jnp
from jax import lax
from jax.experimental import pallas as pl
from jax.experimental.pallas import tpu as pltpu
from jax.experimental.pallas import tpu_sc as plsc

N = 10000
E = 320000
F = 128
R = 16
NBASES = 4

NC = 2
NS = 16
NTILES = NC * NS
EPT = E // NTILES
K = 80
NBT = EPT // K
C = 25
NCHUNK = NBT // C
CE = C * K
NFL = 10
FRPT = N // NFL
ZROWS = 40


def _zr_body(coeff_ref, w_ref, feat_ref, lw_ref, bias_ref, zr_ref, sl_ref,
             wr_all):
    i = pl.program_id(0)

    @pl.when(i == 0)
    def _build_wr_table():
        for rr in range(R):
            wr_all[:, rr * F:(rr + 1) * F] = (
                coeff_ref[rr, 0] * w_ref[0] + coeff_ref[rr, 1] * w_ref[1]
                + coeff_ref[rr, 2] * w_ref[2] + coeff_ref[rr, 3] * w_ref[3])

    wide = jnp.dot(feat_ref[...], wr_all[...],
                   preferred_element_type=jnp.float32)
    for rr in range(R):
        zr_ref[rr] = wide[:, rr * F:(rr + 1) * F]
    sl_ref[...] = bias_ref[...] + jnp.dot(feat_ref[...], lw_ref[...],
                                          preferred_element_type=jnp.float32)


def _build_zr(coeff, W, feat, lw, bias2d):
    bn = 1000
    nb = N // bn
    return pl.pallas_call(
        _zr_body,
        grid=(nb,),
        in_specs=[
            pl.BlockSpec(memory_space=pltpu.SMEM),
            pl.BlockSpec((NBASES, F, F), lambda i: (0, 0, 0)),
            pl.BlockSpec((bn, F), lambda i: (i, 0)),
            pl.BlockSpec((F, F), lambda i: (0, 0)),
            pl.BlockSpec((1, F), lambda i: (0, 0)),
        ],
        out_specs=[
            pl.BlockSpec((R, bn, F), lambda i: (0, i, 0)),
            pl.BlockSpec((bn, F), lambda i: (i, 0)),
        ],
        out_shape=[
            jax.ShapeDtypeStruct((R, N, F), jnp.float32),
            jax.ShapeDtypeStruct((N, F), jnp.float32),
        ],
        scratch_shapes=[pltpu.VMEM((F, R * F), jnp.float32)],
    )(coeff, W, feat, lw, bias2d)


def _sc_agg_body(zr, ei, et, out, ridx_v, et_v, dst_v, rows_v,
                 zbuf, acc, sem, isem):
    c = lax.axis_index("c")
    s = lax.axis_index("s")
    wid = c * NS + s
    base = s * FRPT

    def _stage(ch, pc):
        pltpu.async_copy(ei.at[0].at[wid].at[ch], ridx_v.at[pc], isem)
        pltpu.async_copy(et.at[wid].at[ch], et_v.at[pc], isem)
        pltpu.async_copy(ei.at[1].at[wid].at[ch], dst_v.at[pc], isem)

    def _stage_wait(ch, pc):
        pltpu.make_async_copy(ei.at[0].at[wid].at[ch], ridx_v.at[pc],
                              isem).wait()
        pltpu.make_async_copy(et.at[wid].at[ch], et_v.at[pc], isem).wait()
        pltpu.make_async_copy(ei.at[1].at[wid].at[ch], dst_v.at[pc],
                              isem).wait()
        def _cvt(i, cy):
            for j in range(K // 16):
                sl = pl.ds(j * 16, 16)
                ridx_v[pc, i, sl] = ridx_v[pc, i, sl] + et_v[pc, i, sl] * N
            return cy
        lax.fori_loop(0, C, _cvt, 0)

    _stage(0, 0)

    @pl.when(s < NFL)
    def _zero():
        zero16 = jnp.zeros((16,), jnp.float32)
        def _zrow(i, carry):
            for j in range(F // 16):
                zbuf[i, pl.ds(j * 16, 16)] = zero16
            return carry
        lax.fori_loop(0, ZROWS, _zrow, 0)
        for j in range(FRPT // ZROWS):
            pltpu.sync_copy(zbuf, acc.at[pl.ds(base + j * ZROWS, ZROWS)])

    _stage_wait(0, 0)
    plsc.subcore_barrier()

    def _chunk(ch, carry):
        pc = lax.rem(ch, 2)
        qc = lax.rem(ch + 1, 2)

        @pl.when(ch + 1 < NCHUNK)
        def _prefetch_idx():
            _stage(ch + 1, qc)

        pltpu.async_copy(zr.at[ridx_v.at[pc].at[0]], rows_v.at[0], sem.at[0])

        def _edge(i, cy):
            p = lax.rem(i, 2)
            q = lax.rem(i + 1, 2)

            @pl.when(i + 1 < C)
            def _prefetch():
                pltpu.async_copy(zr.at[ridx_v.at[pc].at[i + 1]], rows_v.at[q],
                                 sem.at[q])
            pltpu.make_async_copy(zr.at[ridx_v.at[pc].at[i]], rows_v.at[p],
                                  sem.at[p]).wait()
            pltpu.sync_copy(rows_v.at[p], acc.at[dst_v.at[pc].at[i]], add=True)
            return cy
        lax.fori_loop(0, C, _edge, 0)

        @pl.when(ch + 1 < NCHUNK)
        def _wait_idx():
            _stage_wait(ch + 1, qc)
        return carry
    lax.fori_loop(0, NCHUNK, _chunk, 0)
    plsc.subcore_barrier()

    @pl.when(s < NFL)
    def _flush():
        pltpu.sync_copy(acc.at[pl.ds(base, FRPT)],
                        out.at[c].at[pl.ds(base, FRPT)])


def _sc_agg(zr2, ei, et):
    mesh = plsc.VectorSubcoreMesh(core_axis_name="c", subcore_axis_name="s")
    fn = functools.partial(
        pl.kernel,
        mesh=mesh,
        out_type=jax.ShapeDtypeStruct((NC, N, F), jnp.float32),
        scratch_types=[
            pltpu.VMEM((2, C, K), jnp.int32),
            pltpu.VMEM((2, C, K), jnp.int32),
            pltpu.VMEM((2, C, K), jnp.int32),
            pltpu.VMEM((2, K, F), jnp.float32),
            pltpu.VMEM((ZROWS, F), jnp.float32),
            pltpu.VMEM_SHARED((N, F), jnp.float32),
            pltpu.SemaphoreType.DMA((2,)),
            pltpu.SemaphoreType.DMA,
        ],
    )(_sc_agg_body)
    return fn(zr2, ei, et)


def _combine_body(p0_ref, p1_ref, sl_ref, out_ref):
    out_ref[...] = p0_ref[0] + p1_ref[0] + sl_ref[...]


def _combine(parts, sl):
    bn = 2000
    return pl.pallas_call(
        _combine_body,
        grid=(N // bn,),
        in_specs=[
            pl.BlockSpec((1, bn, F), lambda i: (0, i, 0)),
            pl.BlockSpec((1, bn, F), lambda i: (1, i, 0)),
            pl.BlockSpec((bn, F), lambda i: (i, 0)),
        ],
        out_specs=pl.BlockSpec((bn, F), lambda i: (i, 0)),
        out_shape=jax.ShapeDtypeStruct((N, F), jnp.float32),
    )(parts, parts, sl)


def kernel(feat, edge_index, etypes, W, coeff, h_bias, loop_weight):
    zr3, sl = _build_zr(coeff, W, feat, loop_weight,
                        h_bias.reshape(1, F))
    zr2 = zr3.reshape(R * N, F)
    eiR = edge_index.reshape(2, NTILES, NCHUNK, C, K)
    etR = etypes.reshape(NTILES, NCHUNK, C, K)
    parts = _sc_agg(zr2, eiR, etR)
    return _combine(parts, sl)
